# Initial kernel scaffold; baseline (speedup 1.0000x reference)
import jax
import jax.numpy as jnp
from jax.experimental import pallas as pl


def kernel(tokens, W_pos):
    raise NotImplementedError("write your pallas kernel here")


# speedup vs baseline: 1.4525x; 1.4525x over previous
import jax
import jax.numpy as jnp
from jax.experimental import pallas as pl


def _bcast_copy(w_ref, o_ref):
    o_ref[...] = jnp.broadcast_to(w_ref[...][None], o_ref.shape)


def kernel(tokens, W_pos):
    batch, seq_len = tokens.shape
    d_model = W_pos.shape[1]
    s_blk = 512
    grid = (seq_len // s_blk,)
    out = pl.pallas_call(
        _bcast_copy,
        grid=grid,
        in_specs=[pl.BlockSpec((s_blk, d_model), lambda i: (i, 0))],
        out_specs=pl.BlockSpec((batch, s_blk, d_model), lambda i: (0, i, 0)),
        out_shape=jax.ShapeDtypeStruct((batch, seq_len, d_model), W_pos.dtype),
    )(W_pos)
    return out


# speedup vs baseline: 1.5037x; 1.0353x over previous
import jax
import jax.numpy as jnp
from jax.experimental import pallas as pl


def _bcast_copy(w_ref, o_ref):
    o_ref[...] = jnp.broadcast_to(w_ref[...][None], o_ref.shape)


def kernel(tokens, W_pos):
    batch, seq_len = tokens.shape
    d_model = W_pos.shape[1]
    s_blk = 1024
    grid = (seq_len // s_blk,)
    out = pl.pallas_call(
        _bcast_copy,
        grid=grid,
        in_specs=[pl.BlockSpec((s_blk, d_model), lambda i: (i, 0))],
        out_specs=pl.BlockSpec((batch, s_blk, d_model), lambda i: (0, i, 0)),
        out_shape=jax.ShapeDtypeStruct((batch, seq_len, d_model), W_pos.dtype),
    )(W_pos)
    return out
